# baseline (device time: 15769 ns/iter reference)
import jax
import jax.numpy as jnp
from jax import lax
from jax.experimental import pallas as pl
from jax.experimental.pallas import tpu as pltpu

N_DEV = 4
HALO = 3
HPAD = 8
BLK = 128


def _silu(a):
    return a * jax.nn.sigmoid(a)


def kernel(x, k):
    b, s, c = x.shape
    taps = k.shape[0]
    nb = s // BLK

    def body(x_ref, tail_ref, ptail_ref, k_ref, out_ref,
             send_buf, recv_buf, halo_buf, send_sem, recv_sem, exit_sem):
        j = pl.program_id(0)
        my = lax.axis_index("i")
        left = (my + N_DEV - 1) % N_DEV
        right = (my + 1) % N_DEV

        def mk_rdma():
            return pltpu.make_async_remote_copy(
                src_ref=send_buf,
                dst_ref=recv_buf,
                send_sem=send_sem,
                recv_sem=recv_sem,
                device_id=(right,),
                device_id_type=pl.DeviceIdType.MESH,
            )

        @pl.when(j == 0)
        def _():
            barrier = pltpu.get_barrier_semaphore()
            for nbr in (left, right):
                pl.semaphore_signal(
                    barrier, inc=1,
                    device_id=(nbr,), device_id_type=pl.DeviceIdType.MESH,
                )
            pl.semaphore_wait(barrier, 2)
            send_buf[:, :, :] = tail_ref[:, :, :]
            mk_rdma().start()

        @pl.when(j < nb - 1)
        def _():
            halo_buf[:, :, :] = ptail_ref[:, :, :]

        @pl.when(j == nb - 1)
        def _():
            rdma = mk_rdma()
            rdma.wait_recv()
            rdma.wait_send()
            hv = recv_buf[:, :, :]
            halo_buf[:, :, :] = jnp.where(my == 0, jnp.zeros_like(hv), hv)

        xv = x_ref[:, :, :].astype(jnp.bfloat16)
        kv = k_ref[:, :].astype(jnp.bfloat16)
        hv = halo_buf[:, HPAD - HALO:, :].astype(jnp.bfloat16)
        hp = jnp.concatenate([hv, xv], axis=1)
        acc = jnp.zeros((b, BLK, c), jnp.bfloat16)
        for t in range(taps):
            acc = acc + hp[:, t:t + BLK, :] * kv[t, :]
        out_ref[:, :, :] = _silu(acc)

        @pl.when(j == nb - 1)
        def _():
            for nbr in (left, right):
                pl.semaphore_signal(
                    exit_sem, inc=1,
                    device_id=(nbr,), device_id_type=pl.DeviceIdType.MESH,
                )
            pl.semaphore_wait(exit_sem, 2)

    grid = (nb,)
    tpb = BLK // HPAD

    return pl.pallas_call(
        body,
        grid=grid,
        out_shape=jax.ShapeDtypeStruct((b, s, c), jnp.bfloat16),
        in_specs=[
            pl.BlockSpec((b, BLK, c), lambda j: (0, (j + 1) % nb, 0)),
            pl.BlockSpec((b, HPAD, c), lambda j: (0, s // HPAD - 1, 0)),
            pl.BlockSpec(
                (b, HPAD, c),
                lambda j: (0, jnp.maximum(((j + 1) % nb) * tpb - 1, 0), 0),
            ),
            pl.BlockSpec((taps, c), lambda j: (0, 0)),
        ],
        out_specs=pl.BlockSpec((b, BLK, c), lambda j: (0, (j + 1) % nb, 0)),
        scratch_shapes=[
            pltpu.VMEM((b, HPAD, c), x.dtype),
            pltpu.VMEM((b, HPAD, c), x.dtype),
            pltpu.VMEM((b, HPAD, c), x.dtype),
            pltpu.SemaphoreType.DMA,
            pltpu.SemaphoreType.DMA,
            pltpu.SemaphoreType.REGULAR,
        ],
        compiler_params=pltpu.CompilerParams(collective_id=0),
    )(x, x, x, k)
